# speedup vs baseline: 7.1852x; 7.1852x over previous
import jax
import jax.numpy as jnp
from jax import lax
from jax.experimental import pallas as pl
from jax.experimental.pallas import tpu as pltpu
from jax.experimental.pallas import tpu_sc as plsc

NC = 2
NS = 16
L = 16
NW = NC * NS

BATCH = 4096
SEQ = 50
OUT_DIM = 128

B_PER_W = BATCH // NW
ROWS_PER_CHUNK = 2
IDX_PER_CHUNK = ROWS_PER_CHUNK * SEQ
N_CHUNKS = B_PER_W // ROWS_PER_CHUNK
N_VREG = OUT_DIM // L


def _sc_kernel(ids_hbm, table_hbm, out_hbm, idx_v, rows_v, out_v, sem):
    wid = lax.axis_index("subcore") * NC + lax.axis_index("core")
    pltpu.sync_copy(ids_hbm.at[pl.ds(wid * N_CHUNKS, N_CHUNKS), :], idx_v)

    @pl.loop(0, N_CHUNKS)
    def _chunk(c):
        pltpu.async_copy(table_hbm.at[idx_v.at[c]], rows_v, sem).wait()
        for r in range(ROWS_PER_CHUNK):
            def seq_body(s, accs):
                base = r * SEQ + s
                return tuple(
                    accs[j] + rows_v[base, pl.ds(j * L, L)]
                    for j in range(N_VREG)
                )
            accs = lax.fori_loop(
                0, SEQ, seq_body,
                tuple(jnp.zeros((L,), jnp.float32) for _ in range(N_VREG)),
            )
            for j in range(N_VREG):
                out_v[c * ROWS_PER_CHUNK + r, pl.ds(j * L, L)] = (
                    accs[j] * (1.0 / SEQ)
                )

    pltpu.sync_copy(out_v, out_hbm.at[pl.ds(wid * B_PER_W, B_PER_W), :])


@jax.jit
def kernel(input_ids, table):
    ids2d = input_ids.reshape(BATCH * SEQ // IDX_PER_CHUNK, IDX_PER_CHUNK)
    mesh = plsc.VectorSubcoreMesh(core_axis_name="core",
                                  subcore_axis_name="subcore")
    run = pl.kernel(
        _sc_kernel,
        out_type=jax.ShapeDtypeStruct((BATCH, OUT_DIM), jnp.float32),
        mesh=mesh,
        scratch_types=[
            pltpu.VMEM((N_CHUNKS, IDX_PER_CHUNK), jnp.int32),
            pltpu.VMEM((IDX_PER_CHUNK, OUT_DIM), jnp.float32),
            pltpu.VMEM((B_PER_W, OUT_DIM), jnp.float32),
            pltpu.SemaphoreType.DMA,
        ],
    )
    return run(ids2d.astype(jnp.int32), table)


# speedup vs baseline: 11.6385x; 1.6198x over previous
import jax
import jax.numpy as jnp
from jax import lax
from jax.experimental import pallas as pl
from jax.experimental.pallas import tpu as pltpu
from jax.experimental.pallas import tpu_sc as plsc

NC = 2
NS = 16
L = 16
NW = NC * NS

BATCH = 4096
SEQ = 50
OUT_DIM = 128

B_PER_W = BATCH // NW
ROWS_PER_CHUNK = 2
IDX_PER_CHUNK = ROWS_PER_CHUNK * SEQ
N_CHUNKS = B_PER_W // ROWS_PER_CHUNK
N_VREG = OUT_DIM // L


def _accumulate(rows_v, out_v, c):
    for r in range(ROWS_PER_CHUNK):
        def seq_body(s, accs):
            base = r * SEQ + s
            return tuple(
                accs[j] + rows_v[base, pl.ds(j * L, L)]
                for j in range(N_VREG)
            )
        accs = lax.fori_loop(
            0, SEQ, seq_body,
            tuple(jnp.zeros((L,), jnp.float32) for _ in range(N_VREG)),
        )
        for j in range(N_VREG):
            out_v[c * ROWS_PER_CHUNK + r, pl.ds(j * L, L)] = (
                accs[j] * (1.0 / SEQ)
            )


def _sc_kernel(ids_hbm, table_hbm, out_hbm, idx_v, rows0, rows1, out_v,
               sem0, sem1):
    wid = lax.axis_index("subcore") * NC + lax.axis_index("core")
    pltpu.sync_copy(ids_hbm.at[pl.ds(wid * N_CHUNKS, N_CHUNKS), :], idx_v)

    pltpu.async_copy(table_hbm.at[idx_v.at[0]], rows0, sem0)

    @pl.loop(0, N_CHUNKS, step=2)
    def _chunk(c):
        pltpu.async_copy(table_hbm.at[idx_v.at[c + 1]], rows1, sem1)
        pltpu.make_async_copy(table_hbm.at[idx_v.at[c]], rows0, sem0).wait()
        _accumulate(rows0, out_v, c)

        @pl.when(c + 2 < N_CHUNKS)
        def _():
            pltpu.async_copy(table_hbm.at[idx_v.at[c + 2]], rows0, sem0)

        pltpu.make_async_copy(table_hbm.at[idx_v.at[c + 1]], rows1,
                              sem1).wait()
        _accumulate(rows1, out_v, c + 1)

    pltpu.sync_copy(out_v, out_hbm.at[pl.ds(wid * B_PER_W, B_PER_W), :])


@jax.jit
def kernel(input_ids, table):
    ids2d = input_ids.reshape(BATCH * SEQ // IDX_PER_CHUNK, IDX_PER_CHUNK)
    mesh = plsc.VectorSubcoreMesh(core_axis_name="core",
                                  subcore_axis_name="subcore")
    run = pl.kernel(
        _sc_kernel,
        out_type=jax.ShapeDtypeStruct((BATCH, OUT_DIM), jnp.float32),
        mesh=mesh,
        scratch_types=[
            pltpu.VMEM((N_CHUNKS, IDX_PER_CHUNK), jnp.int32),
            pltpu.VMEM((IDX_PER_CHUNK, OUT_DIM), jnp.float32),
            pltpu.VMEM((IDX_PER_CHUNK, OUT_DIM), jnp.float32),
            pltpu.VMEM((B_PER_W, OUT_DIM), jnp.float32),
            pltpu.SemaphoreType.DMA,
            pltpu.SemaphoreType.DMA,
        ],
    )
    return run(ids2d.astype(jnp.int32), table)


# speedup vs baseline: 15.5820x; 1.3388x over previous
import jax
import jax.numpy as jnp
from jax import lax
from jax.experimental import pallas as pl
from jax.experimental.pallas import tpu as pltpu
from jax.experimental.pallas import tpu_sc as plsc

NC = 2
NS = 16
L = 16
NW = NC * NS

BATCH = 4096
SEQ = 50
OUT_DIM = 128

B_PER_W = BATCH // NW
ROWS_PER_CHUNK = 2
IDX_PER_CHUNK = ROWS_PER_CHUNK * SEQ
N_CHUNKS = B_PER_W // ROWS_PER_CHUNK
N_VREG = OUT_DIM // L


def _accumulate(rows_v, out_v, c):
    for r in range(ROWS_PER_CHUNK):
        def seq_body(s, accs):
            base = r * SEQ + s
            return tuple(
                accs[j] + rows_v[base, pl.ds(j * L, L)]
                for j in range(N_VREG)
            )
        accs = lax.fori_loop(
            0, SEQ, seq_body,
            tuple(jnp.zeros((L,), jnp.float32) for _ in range(N_VREG)),
        )
        for j in range(N_VREG):
            out_v[c * ROWS_PER_CHUNK + r, pl.ds(j * L, L)] = (
                accs[j] * (1.0 / SEQ)
            )


NBUF = 4


def _sc_kernel(ids_hbm, table_hbm, out_hbm, idx_v, rows, sems, out_v):
    wid = lax.axis_index("subcore") * NC + lax.axis_index("core")
    pltpu.sync_copy(ids_hbm.at[pl.ds(wid * N_CHUNKS, N_CHUNKS), :], idx_v)

    for b in range(NBUF - 1):
        pltpu.async_copy(table_hbm.at[idx_v.at[b]], rows[b], sems[b])

    @pl.loop(0, N_CHUNKS, step=NBUF)
    def _chunk(c):
        for b in range(NBUF):
            nxt = c + b + NBUF - 1
            nb = (b + NBUF - 1) % NBUF

            @pl.when(nxt < N_CHUNKS)
            def _():
                pltpu.async_copy(table_hbm.at[idx_v.at[nxt]], rows[nb],
                                 sems[nb])

            pltpu.make_async_copy(table_hbm.at[idx_v.at[c + b]], rows[b],
                                  sems[b]).wait()
            _accumulate(rows[b], out_v, c + b)

    pltpu.sync_copy(out_v, out_hbm.at[pl.ds(wid * B_PER_W, B_PER_W), :])


@jax.jit
def kernel(input_ids, table):
    ids2d = input_ids.reshape(BATCH * SEQ // IDX_PER_CHUNK, IDX_PER_CHUNK)
    mesh = plsc.VectorSubcoreMesh(core_axis_name="core",
                                  subcore_axis_name="subcore")
    run = pl.kernel(
        _sc_kernel,
        out_type=jax.ShapeDtypeStruct((BATCH, OUT_DIM), jnp.float32),
        mesh=mesh,
        scratch_types=[
            pltpu.VMEM((N_CHUNKS, IDX_PER_CHUNK), jnp.int32),
            [pltpu.VMEM((IDX_PER_CHUNK, OUT_DIM), jnp.float32)
             for _ in range(NBUF)],
            [pltpu.SemaphoreType.DMA for _ in range(NBUF)],
            pltpu.VMEM((B_PER_W, OUT_DIM), jnp.float32),
        ],
    )
    return run(ids2d.astype(jnp.int32), table)


# speedup vs baseline: 15.6813x; 1.0064x over previous
import jax
import jax.numpy as jnp
from jax import lax
from jax.experimental import pallas as pl
from jax.experimental.pallas import tpu as pltpu
from jax.experimental.pallas import tpu_sc as plsc

NC = 2
NS = 16
L = 16
NW = NC * NS

BATCH = 4096
SEQ = 50
OUT_DIM = 128

B_PER_W = BATCH // NW
ROWS_PER_CHUNK = 2
IDX_PER_CHUNK = ROWS_PER_CHUNK * SEQ
N_CHUNKS = B_PER_W // ROWS_PER_CHUNK
N_VREG = OUT_DIM // L


def _accumulate(rows_v, out_v, c):
    for r in range(ROWS_PER_CHUNK):
        def seq_body(s, accs):
            base = r * SEQ + s
            return tuple(
                accs[j] + rows_v[base, pl.ds(j * L, L)]
                for j in range(N_VREG)
            )
        accs = lax.fori_loop(
            0, SEQ, seq_body,
            tuple(jnp.zeros((L,), jnp.float32) for _ in range(N_VREG)),
        )
        for j in range(N_VREG):
            out_v[c * ROWS_PER_CHUNK + r, pl.ds(j * L, L)] = (
                accs[j] * (1.0 / SEQ)
            )


NBUF = 6


def _sc_kernel(ids_hbm, table_hbm, out_hbm, idx_v, rows, sems, out_v):
    wid = lax.axis_index("subcore") * NC + lax.axis_index("core")
    pltpu.sync_copy(ids_hbm.at[pl.ds(wid * N_CHUNKS, N_CHUNKS), :], idx_v)

    for b in range(NBUF - 1):
        pltpu.async_copy(table_hbm.at[idx_v.at[b]], rows[b], sems[b])

    @pl.loop(0, N_CHUNKS, step=NBUF)
    def _chunk(c):
        for b in range(NBUF):
            nxt = c + b + NBUF - 1
            nb = (b + NBUF - 1) % NBUF

            @pl.when(nxt < N_CHUNKS)
            def _():
                pltpu.async_copy(table_hbm.at[idx_v.at[nxt]], rows[nb],
                                 sems[nb])

            @pl.when(c + b < N_CHUNKS)
            def _():
                pltpu.make_async_copy(table_hbm.at[idx_v.at[c + b]], rows[b],
                                      sems[b]).wait()
                _accumulate(rows[b], out_v, c + b)

    pltpu.sync_copy(out_v, out_hbm.at[pl.ds(wid * B_PER_W, B_PER_W), :])


@jax.jit
def kernel(input_ids, table):
    ids2d = input_ids.reshape(BATCH * SEQ // IDX_PER_CHUNK, IDX_PER_CHUNK)
    mesh = plsc.VectorSubcoreMesh(core_axis_name="core",
                                  subcore_axis_name="subcore")
    run = pl.kernel(
        _sc_kernel,
        out_type=jax.ShapeDtypeStruct((BATCH, OUT_DIM), jnp.float32),
        mesh=mesh,
        scratch_types=[
            pltpu.VMEM((N_CHUNKS, IDX_PER_CHUNK), jnp.int32),
            [pltpu.VMEM((IDX_PER_CHUNK, OUT_DIM), jnp.float32)
             for _ in range(NBUF)],
            [pltpu.SemaphoreType.DMA for _ in range(NBUF)],
            pltpu.VMEM((B_PER_W, OUT_DIM), jnp.float32),
        ],
    )
    return run(ids2d.astype(jnp.int32), table)
